# baseline (speedup 1.0000x reference)
import jax
import jax.numpy as jnp
from jax import lax
from jax.experimental import pallas as pl
from jax.experimental.pallas import tpu as pltpu

STRIDE_F = 8.0
N_CLS_K = 80
NB, NH, NW = 8, 128, 128
NLOC = NH * NW
NSPLIT = 8
R = NLOC // NSPLIT
BROWS = (NLOC * 4 // 128) // NSPLIT


def _sigmoid(x):
    return 1.0 / (1.0 + jnp.exp(-x))


def _body(hw_ref, bbox_ref, center_ref, cls_ref, xywh_ref, idx_ref, conf_ref):
    r = pl.program_id(1)

    logits = cls_ref[0]
    m = jnp.max(logits, axis=1, keepdims=True)
    ii = lax.broadcasted_iota(jnp.int32, (R, N_CLS_K), 1)
    idx = jnp.min(jnp.where(logits == m, ii, N_CLS_K), axis=1, keepdims=True)
    idx_ref[0] = idx

    c = center_ref[0]
    conf_ref[0] = jnp.sqrt(_sigmoid(c) * _sigmoid(m))

    v = bbox_ref[0]
    ri = lax.broadcasted_iota(jnp.int32, (BROWS, 128), 0)
    li = lax.broadcasted_iota(jnp.int32, (BROWS, 128), 1)
    e = (r * BROWS + ri) * 128 + li
    loc = e >> 2
    chan = li & 3
    xf = (loc & (NW - 1)).astype(jnp.float32) * STRIDE_F + STRIDE_F / 2.0
    yf = (loc >> 7).astype(jnp.float32) * STRIDE_F + STRIDE_F / 2.0
    is_x = (chan & 1) == 0
    coord = jnp.where(is_x, xf, yf)
    w_f = hw_ref[1].astype(jnp.float32)
    h_f = hw_ref[0].astype(jnp.float32)
    bound = jnp.where(is_x, w_f, h_f)
    lo_half = chan < 2
    sgn = jnp.where(lo_half, -1.0, 1.0)
    p = jnp.exp(v) * STRIDE_F
    cl = jnp.clip(coord + sgn * p, 0.0, bound)
    partner = jnp.where(lo_half, jnp.roll(cl, -2, axis=1), jnp.roll(cl, 2, axis=1))
    xywh_ref[0] = jnp.where(lo_half, (cl + partner) * 0.5, cl - partner)


def kernel(bbox, center, cls_logits, img_h, img_w):
    nB, nH, nW, _ = bbox.shape
    bbox_i = bbox.reshape(nB, NLOC * 4 // 128, 128)
    center_i = center.reshape(nB, NLOC, 1)
    cls_i = cls_logits.reshape(nB, NLOC, N_CLS_K)
    hw = jnp.stack([jnp.asarray(img_h, jnp.int32), jnp.asarray(img_w, jnp.int32)])

    grid = (nB, NSPLIT)
    xywh_i, idx, conf = pl.pallas_call(
        _body,
        grid=grid,
        in_specs=[
            pl.BlockSpec(memory_space=pltpu.SMEM),
            pl.BlockSpec((1, BROWS, 128), lambda b, r: (b, r, 0)),
            pl.BlockSpec((1, R, 1), lambda b, r: (b, r, 0)),
            pl.BlockSpec((1, R, N_CLS_K), lambda b, r: (b, r, 0)),
        ],
        out_specs=[
            pl.BlockSpec((1, BROWS, 128), lambda b, r: (b, r, 0)),
            pl.BlockSpec((1, R, 1), lambda b, r: (b, r, 0)),
            pl.BlockSpec((1, R, 1), lambda b, r: (b, r, 0)),
        ],
        out_shape=[
            jax.ShapeDtypeStruct((nB, NLOC * 4 // 128, 128), jnp.float32),
            jax.ShapeDtypeStruct((nB, NLOC, 1), jnp.int32),
            jax.ShapeDtypeStruct((nB, NLOC, 1), jnp.float32),
        ],
        compiler_params=pltpu.CompilerParams(
            dimension_semantics=("parallel", "parallel"),
        ),
    )(hw, bbox_i, center_i, cls_i)

    return (
        xywh_i.reshape(nB, NLOC, 4),
        idx.reshape(nB, NLOC),
        conf.reshape(nB, NLOC),
    )


# speedup vs baseline: 1.7151x; 1.7151x over previous
import jax
import jax.numpy as jnp
from jax import lax
from jax.experimental import pallas as pl
from jax.experimental.pallas import tpu as pltpu

STRIDE_F = 8.0
N_CLS_K = 80
NB, NH, NW = 8, 128, 128
NLOC = NH * NW
NSPLIT = 8
R = NLOC // NSPLIT
BROWS = (NLOC * 4 // 128) // NSPLIT


def _sigmoid(x):
    return 1.0 / (1.0 + jnp.exp(-x))


def _body(hw_ref, bbox_ref, center_ref, cls_ref, xywh_ref, idx_ref, conf_ref):
    r = pl.program_id(1)

    logits = cls_ref[0]
    xt = logits.T
    m = jnp.max(xt, axis=0, keepdims=True)
    ii = lax.broadcasted_iota(jnp.int32, (N_CLS_K, R), 0)
    idx = jnp.min(jnp.where(xt == m, ii, N_CLS_K), axis=0, keepdims=True)
    idx_ref[0] = idx

    c = center_ref[0]
    conf_ref[0] = jnp.sqrt(_sigmoid(c) * _sigmoid(m))

    v = bbox_ref[0]
    ri = lax.broadcasted_iota(jnp.int32, (BROWS, 128), 0)
    li = lax.broadcasted_iota(jnp.int32, (BROWS, 128), 1)
    t = ri * 32 + (li >> 2)
    xg = t & (NW - 1)
    yg = (t >> 7) + r * (R // NW)
    chan = li & 3
    xf = xg.astype(jnp.float32) * STRIDE_F + STRIDE_F / 2.0
    yf = yg.astype(jnp.float32) * STRIDE_F + STRIDE_F / 2.0
    is_x = (chan & 1) == 0
    coord = jnp.where(is_x, xf, yf)
    w_f = hw_ref[1].astype(jnp.float32)
    h_f = hw_ref[0].astype(jnp.float32)
    bound = jnp.where(is_x, w_f, h_f)
    lo_half = chan < 2
    sgn = jnp.where(lo_half, -1.0, 1.0)
    p = jnp.exp(v) * STRIDE_F
    cl = jnp.clip(coord + sgn * p, 0.0, bound)
    partner = jnp.where(lo_half, jnp.roll(cl, -2, axis=1), jnp.roll(cl, 2, axis=1))
    xywh_ref[0] = jnp.where(lo_half, (cl + partner) * 0.5, cl - partner)


def kernel(bbox, center, cls_logits, img_h, img_w):
    nB, nH, nW, _ = bbox.shape
    bbox_i = bbox.reshape(nB, NLOC * 4 // 128, 128)
    center_i = center.reshape(nB * NSPLIT, 1, R)
    cls_i = cls_logits.reshape(nB, NLOC, N_CLS_K)
    hw = jnp.stack([jnp.asarray(img_h, jnp.int32), jnp.asarray(img_w, jnp.int32)])

    grid = (nB, NSPLIT)
    xywh_i, idx, conf = pl.pallas_call(
        _body,
        grid=grid,
        in_specs=[
            pl.BlockSpec(memory_space=pltpu.SMEM),
            pl.BlockSpec((1, BROWS, 128), lambda b, r: (b, r, 0)),
            pl.BlockSpec((1, 1, R), lambda b, r: (b * NSPLIT + r, 0, 0)),
            pl.BlockSpec((1, R, N_CLS_K), lambda b, r: (b, r, 0)),
        ],
        out_specs=[
            pl.BlockSpec((1, BROWS, 128), lambda b, r: (b, r, 0)),
            pl.BlockSpec((1, 1, R), lambda b, r: (b * NSPLIT + r, 0, 0)),
            pl.BlockSpec((1, 1, R), lambda b, r: (b * NSPLIT + r, 0, 0)),
        ],
        out_shape=[
            jax.ShapeDtypeStruct((nB, NLOC * 4 // 128, 128), jnp.float32),
            jax.ShapeDtypeStruct((nB * NSPLIT, 1, R), jnp.int32),
            jax.ShapeDtypeStruct((nB * NSPLIT, 1, R), jnp.float32),
        ],
        compiler_params=pltpu.CompilerParams(
            dimension_semantics=("parallel", "parallel"),
        ),
    )(hw, bbox_i, center_i, cls_i)

    return (
        xywh_i.reshape(nB, NLOC, 4),
        idx.reshape(nB, NLOC),
        conf.reshape(nB, NLOC),
    )
